# baseline (speedup 1.0000x reference)
import functools

import jax
import jax.numpy as jnp
from jax import lax
from jax.experimental import pallas as pl
from jax.experimental.pallas import tpu as pltpu
from jax.experimental.pallas import tpu_sc as plsc

_N = 10000
_E = 320000
_D = 128
_R = 8
_NC = 2
_NS = 16
_K = 80
_G = _K // 16
_NP = 10240
_NPT = _NP // _NS
_CC = _E // (_NS * _K)
_MC = _E // (_NC * _NS * _K)
_ET = _E // (_NC * _NS)

_mesh = plsc.VectorSubcoreMesh(core_axis_name="c", subcore_axis_name="s",
                               num_cores=_NC, num_subcores=_NS)
_sc_params = pltpu.CompilerParams(needs_layout_passes=False,
                                  use_tc_tiling_on_sc=False)


def _rel_matmul(h, w_all):
    n, d = h.shape
    kk = w_all.shape[0]
    blk = 1000
    nb = n // blk

    def body(h_ref, w_ref, o_ref):
        o_ref[...] = jnp.dot(h_ref[...].astype(jnp.bfloat16),
                             w_ref[0].astype(jnp.bfloat16),
                             preferred_element_type=jnp.float32)

    return pl.pallas_call(
        body,
        grid=(kk, nb),
        in_specs=[
            pl.BlockSpec((blk, d), lambda k, b: (b, 0)),
            pl.BlockSpec((1, d, d), lambda k, b: (k, 0, 0)),
        ],
        out_specs=pl.BlockSpec((blk, d), lambda k, b: (k * nb + b, 0)),
        out_shape=jax.ShapeDtypeStruct((kk * n, d), jnp.float32),
    )(h, w_all)


def _combine(accp, hw_full, bias, h_prev):
    n, d = h_prev.shape
    blk = 1000
    nb = n // blk
    self_blk0 = _R * n // blk

    def body(a0, a1, hs, b_ref, hp, o_ref):
        u = jnp.maximum(a0[0] + a1[0] + hs[...] + b_ref[...], 0.0)
        o_ref[...] = u + hp[...]

    row = pl.BlockSpec((blk, d), lambda i: (i, 0))
    return pl.pallas_call(
        body,
        grid=(nb,),
        in_specs=[
            pl.BlockSpec((1, blk, d), lambda i: (0, i, 0)),
            pl.BlockSpec((1, blk, d), lambda i: (1, i, 0)),
            pl.BlockSpec((blk, d), lambda i: (self_blk0 + i, 0)),
            pl.BlockSpec((1, d), lambda i: (0, 0)),
            row,
        ],
        out_specs=row,
        out_shape=jax.ShapeDtypeStruct((n, d), jnp.float32),
    )(accp, accp, hw_full, bias.reshape(1, d), h_prev)


def _combine_matmul(accp, hw1_full, bias, h_prev, w_all2):
    n, d = h_prev.shape
    kk = w_all2.shape[0]
    blk = 1000
    nb = n // blk
    self_blk0 = _R * n // blk

    def body(a0, a1, hs, b_ref, hp, w_ref, h1_ref, o_ref, h1_scr):
        k = pl.program_id(0)
        b = pl.program_id(1)

        @pl.when(k == 0)
        def _():
            u = jnp.maximum(a0[0] + a1[0] + hs[...] + b_ref[...], 0.0)
            h1_scr[pl.ds(b * blk, blk), :] = u + hp[...]

        h1blk = h1_scr[pl.ds(b * blk, blk), :]
        h1_ref[...] = h1blk
        o_ref[...] = jnp.dot(h1blk.astype(jnp.bfloat16),
                             w_ref[0].astype(jnp.bfloat16),
                             preferred_element_type=jnp.float32)

    return pl.pallas_call(
        body,
        grid=(kk, nb),
        in_specs=[
            pl.BlockSpec((1, blk, d), lambda k, b: (0, b, 0)),
            pl.BlockSpec((1, blk, d), lambda k, b: (1, b, 0)),
            pl.BlockSpec((blk, d), lambda k, b: (self_blk0 + b, 0)),
            pl.BlockSpec((1, d), lambda k, b: (0, 0)),
            pl.BlockSpec((blk, d), lambda k, b: (b, 0)),
            pl.BlockSpec((1, d, d), lambda k, b: (k, 0, 0)),
        ],
        out_specs=[
            pl.BlockSpec((blk, d), lambda k, b: (b, 0)),
            pl.BlockSpec((blk, d), lambda k, b: (k * nb + b, 0)),
        ],
        out_shape=[
            jax.ShapeDtypeStruct((n, d), jnp.float32),
            jax.ShapeDtypeStruct((kk * n, d), jnp.float32),
        ],
        scratch_shapes=[pltpu.VMEM((n, d), jnp.float32)],
    )(accp, accp, hw1_full, bias.reshape(1, d), h_prev, w_all2)


def _head_matmul(d1, d2, ctx, wh1, bh1, wh2_row, bh2):
    b, d = d1.shape
    cd = ctx.shape[1]
    h = wh1.shape[1]

    def body(d1r, d2r, cr, w1r, b1r, w2r, b2r, o_ref):
        w1 = w1r[...]
        z = jnp.dot(d1r[...], w1[0:d], preferred_element_type=jnp.float32)
        z = z + jnp.dot(d2r[...], w1[d:2 * d],
                        preferred_element_type=jnp.float32)
        z = z + jnp.dot(cr[...], w1[2 * d:2 * d + cd],
                        preferred_element_type=jnp.float32)
        hdd = jnp.maximum(z + b1r[...], 0.0)
        o_ref[...] = (jnp.sum(hdd * w2r[...], axis=1, keepdims=True)
                      + b2r[...])

    return pl.pallas_call(
        body,
        out_shape=jax.ShapeDtypeStruct((b, 1), jnp.float32),
    )(d1, d2, ctx, wh1, bh1.reshape(1, h), wh2_row, bh2.reshape(1, 1))


def _zero_vmem_rows(ref, rows, width):
    zeros16 = jnp.zeros((16,), jnp.float32)

    @pl.loop(0, rows)
    def _(i):
        for j in range(width // 16):
            ref[i, pl.ds(j * 16, 16)] = zeros16


@functools.partial(
    pl.kernel,
    out_type=jax.ShapeDtypeStruct((_E,), jnp.float32),
    mesh=_mesh,
    compiler_params=_sc_params,
    scratch_types=[
        pltpu.VMEM_SHARED((_NP, 16), jnp.float32),
        [pltpu.VMEM((_K, 16), jnp.float32)] * 2,
        [pltpu.VMEM((_K,), jnp.int32)] * 2,
        [pltpu.VMEM((_K,), jnp.int32)] * 2,
        [pltpu.VMEM((_K,), jnp.int32)] * 2,
        [pltpu.VMEM((_K, 16), jnp.float32)] * 2,
        [pltpu.VMEM((_K,), jnp.float32)] * 2,
        pltpu.VMEM((_NPT, 16), jnp.float32),
        [pltpu.SemaphoreType.DMA] * 2,
        [pltpu.SemaphoreType.DMA] * 2,
        [pltpu.SemaphoreType.DMA] * 2,
    ],
)
def _weights_sc(dst_hbm, typ_hbm, w_hbm,
                cnt_sh, oh_v, dst_v, cdst_v, typ_v, inv_v, w_v, cs_v,
                sem_i, sem_w, sem_o):
    c = lax.axis_index("c")
    s = lax.axis_index("s")
    iota = lax.iota(jnp.int32, 16)
    ones16 = jnp.ones((16,), jnp.float32)
    zeros16 = jnp.zeros((16,), jnp.float32)

    _zero_vmem_rows(cs_v, _NPT, 16)
    pltpu.sync_copy(cs_v, cnt_sh.at[pl.ds(s * _NPT, _NPT)])
    plsc.subcore_barrier()

    def issue_cnt(t, b):
        base = (s * _CC + t) * _K
        pltpu.async_copy(dst_hbm.at[pl.ds(base, _K)], dst_v[b], sem_i[b])
        pltpu.async_copy(typ_hbm.at[pl.ds(base, _K)], typ_v[b], sem_i[b])

    def wait_cnt(t, b):
        base = (s * _CC + t) * _K
        pltpu.make_async_copy(dst_hbm.at[pl.ds(base, _K)], dst_v[b],
                              sem_i[b]).wait()
        pltpu.make_async_copy(typ_hbm.at[pl.ds(base, _K)], typ_v[b],
                              sem_i[b]).wait()

    def do_cnt(t, b):
        @pl.when(t >= 2)
        def _():
            pltpu.make_async_copy(oh_v[b], cnt_sh.at[cdst_v[b]],
                                  sem_o[b]).wait()

        for g in range(_G):
            sl = pl.ds(g * 16, 16)
            cdst_v[b][sl] = dst_v[b][sl]

        @pl.loop(0, _K, unroll=8)
        def _(i):
            tyb = plsc.load_gather(typ_v[b], [jnp.full((16,), i, jnp.int32)])
            oh_v[b][i, pl.ds(0, 16)] = jnp.where(iota == tyb, 1.0, 0.0)

        pltpu.async_copy(oh_v[b], cnt_sh.at[cdst_v[b]], sem_o[b], add=True)

    issue_cnt(0, 0)

    @pl.loop(0, _CC // 2)
    def _(u):
        for binner in range(2):
            ct = 2 * u + binner
            b = binner
            wait_cnt(ct, b)

            @pl.when(ct + 1 < _CC)
            def _():
                issue_cnt(ct + 1, 1 - b)

            do_cnt(ct, b)

    pltpu.make_async_copy(oh_v[0], cnt_sh.at[cdst_v[0]], sem_o[0]).wait()
    pltpu.make_async_copy(oh_v[1], cnt_sh.at[cdst_v[1]], sem_o[1]).wait()
    plsc.subcore_barrier()

    pltpu.sync_copy(cnt_sh.at[pl.ds(s * _NPT, _NPT)], cs_v)

    @pl.loop(0, _NPT)
    def _(i):
        v = cs_v[i, pl.ds(0, 16)]
        cs_v[i, pl.ds(0, 16)] = 1.0 / jnp.maximum(v, 1.0)

    pltpu.sync_copy(cs_v, cnt_sh.at[pl.ds(s * _NPT, _NPT)])
    plsc.subcore_barrier()

    half = c * (_E // _NC) + s * _ET

    def issue_w(t, b):
        base = half + t * _K
        pltpu.async_copy(dst_hbm.at[pl.ds(base, _K)], dst_v[b], sem_i[b])
        pltpu.async_copy(typ_hbm.at[pl.ds(base, _K)], typ_v[b], sem_i[b])

    def wait_w(t, b):
        base = half + t * _K
        pltpu.make_async_copy(dst_hbm.at[pl.ds(base, _K)], dst_v[b],
                              sem_i[b]).wait()
        pltpu.make_async_copy(typ_hbm.at[pl.ds(base, _K)], typ_v[b],
                              sem_i[b]).wait()

    def do_w(t, b):
        pltpu.sync_copy(cnt_sh.at[dst_v[b]], inv_v[b])

        @pl.when(t >= 2)
        def _():
            pltpu.make_async_copy(
                w_v[b], w_hbm.at[pl.ds(half + (t - 2) * _K, _K)],
                sem_w[b]).wait()

        for g in range(_G):
            rows = iota + g * 16
            ty = typ_v[b][pl.ds(g * 16, 16)]
            w_v[b][pl.ds(g * 16, 16)] = plsc.load_gather(inv_v[b], [rows, ty])
        pltpu.async_copy(w_v[b], w_hbm.at[pl.ds(half + t * _K, _K)],
                         sem_w[b])

    issue_w(0, 0)

    @pl.loop(0, _MC // 2)
    def _(u):
        for binner in range(2):
            ct = 2 * u + binner
            b = binner
            wait_w(ct, b)

            @pl.when(ct + 1 < _MC)
            def _():
                issue_w(ct + 1, 1 - b)

            do_w(ct, b)

    wait_w(_MC - 1, 0)
    do_w(_MC - 1, 0)

    pltpu.make_async_copy(w_v[1], w_hbm.at[pl.ds(half + (_MC - 2) * _K, _K)],
                          sem_w[1]).wait()
    pltpu.make_async_copy(w_v[0], w_hbm.at[pl.ds(half + (_MC - 1) * _K, _K)],
                          sem_w[0]).wait()


@functools.partial(
    pl.kernel,
    out_type=jax.ShapeDtypeStruct((_NC, _NP, _D), jnp.float32),
    mesh=_mesh,
    compiler_params=_sc_params,
    scratch_types=[
        pltpu.VMEM_SHARED((_NP, _D), jnp.float32),
        [pltpu.VMEM((_K,), jnp.int32)] * 3,
        [pltpu.VMEM((_K,), jnp.int32)] * 3,
        [pltpu.VMEM((_K,), jnp.int32)] * 3,
        [pltpu.VMEM((_K,), jnp.int32)] * 3,
        [pltpu.VMEM((_K,), jnp.float32)] * 3,
        [pltpu.VMEM((_K, _D), jnp.float32)] * 3,
        [pltpu.SemaphoreType.DMA] * 3,
        [pltpu.SemaphoreType.DMA] * 3,
        [pltpu.SemaphoreType.DMA] * 3,
    ],
)
def _layer_sc(table_hbm, src_hbm, dst_hbm, typ_hbm, w_hbm, acc_hbm,
              acc_sh, dst_v, sdst_v, typ_v, idx_v, w_v, rows_v,
              sem_i, sem_g, sem_s):
    c = lax.axis_index("c")
    s = lax.axis_index("s")

    _zero_vmem_rows(rows_v[0], _K, _D)

    @pl.loop(0, _NPT // _K)
    def _(k):
        pltpu.sync_copy(rows_v[0], acc_sh.at[pl.ds(s * _NPT + k * _K, _K)])

    plsc.subcore_barrier()

    half = c * (_E // _NC) + s * _ET

    def issue_idx(t, b):
        base = half + t * _K
        pltpu.async_copy(src_hbm.at[pl.ds(base, _K)], idx_v[b], sem_i[b])
        pltpu.async_copy(dst_hbm.at[pl.ds(base, _K)], dst_v[b], sem_i[b])
        pltpu.async_copy(typ_hbm.at[pl.ds(base, _K)], typ_v[b], sem_i[b])
        pltpu.async_copy(w_hbm.at[pl.ds(base, _K)], w_v[b], sem_i[b])

    def wait_idx(t, b):
        base = half + t * _K
        pltpu.make_async_copy(src_hbm.at[pl.ds(base, _K)], idx_v[b],
                              sem_i[b]).wait()
        pltpu.make_async_copy(dst_hbm.at[pl.ds(base, _K)], dst_v[b],
                              sem_i[b]).wait()
        pltpu.make_async_copy(typ_hbm.at[pl.ds(base, _K)], typ_v[b],
                              sem_i[b]).wait()
        pltpu.make_async_copy(w_hbm.at[pl.ds(base, _K)], w_v[b],
                              sem_i[b]).wait()

    def start_gather(t, b):
        @pl.when(t >= 3)
        def _():
            pltpu.make_async_copy(rows_v[b], acc_sh.at[sdst_v[b]],
                                  sem_s[b]).wait()

        for g in range(_G):
            sl = pl.ds(g * 16, 16)
            idx_v[b][sl] = typ_v[b][sl] * _N + idx_v[b][sl]
        pltpu.async_copy(table_hbm.at[idx_v[b]], rows_v[b], sem_g[b])

    def finish(b):
        pltpu.make_async_copy(table_hbm.at[idx_v[b]], rows_v[b],
                              sem_g[b]).wait()

        for g in range(_G):
            sl = pl.ds(g * 16, 16)
            sdst_v[b][sl] = dst_v[b][sl]

        @pl.loop(0, _K, unroll=16)
        def _(i):
            wb = plsc.load_gather(w_v[b], [jnp.full((16,), i, jnp.int32)])
            for j in range(_D // 16):
                sl = pl.ds(j * 16, 16)
                rows_v[b][i, sl] = rows_v[b][i, sl] * wb

        pltpu.async_copy(rows_v[b], acc_sh.at[sdst_v[b]], sem_s[b],
                         add=True)

    issue_idx(0, 0)
    wait_idx(0, 0)
    start_gather(0, 0)
    issue_idx(1, 1)
    wait_idx(1, 1)
    start_gather(1, 1)
    issue_idx(2, 2)

    @pl.loop(0, (_MC - 2) // 3)
    def _(u):
        for j in range(3):
            ct = 2 + 3 * u + j
            b = (2 + j) % 3
            wait_idx(ct, b)
            start_gather(ct, b)
            finish(j)

            @pl.when(ct + 1 < _MC)
            def _():
                issue_idx(ct + 1, j)

    finish((_MC - 2) % 3)
    finish((_MC - 1) % 3)

    for b in range(3):
        pltpu.make_async_copy(rows_v[b], acc_sh.at[sdst_v[b]],
                              sem_s[b]).wait()

    plsc.subcore_barrier()

    @pl.loop(0, _NPT // _K)
    def _(k):
        row0 = s * _NPT + k * _K
        pltpu.sync_copy(acc_sh.at[pl.ds(row0, _K)], rows_v[0])
        pltpu.sync_copy(rows_v[0], acc_hbm.at[c, pl.ds(row0, _K)])


_BPT = 1024 // (_NC * _NS)


@functools.partial(
    pl.kernel,
    out_type=[
        jax.ShapeDtypeStruct((1024, _D), jnp.float32),
        jax.ShapeDtypeStruct((1024, _D), jnp.float32),
        jax.ShapeDtypeStruct((1024, 64), jnp.float32),
    ],
    mesh=_mesh,
    compiler_params=_sc_params,
    scratch_types=[
        pltpu.VMEM((_BPT,), jnp.int32),
        pltpu.VMEM((_BPT, _D), jnp.float32),
        pltpu.VMEM((_BPT, 64), jnp.float32),
        pltpu.SemaphoreType.DMA,
    ],
)
def _head_gather(h2_hbm, ctx_hbm, id1_hbm, id2_hbm, id3_hbm,
                 o1_hbm, o2_hbm, o3_hbm, idx_v, rows_v, ctx_v, sem):
    c = lax.axis_index("c")
    s = lax.axis_index("s")
    base = (s * _NC + c) * _BPT

    pltpu.sync_copy(id1_hbm.at[pl.ds(base, _BPT)], idx_v)
    pltpu.async_copy(h2_hbm.at[idx_v], rows_v, sem).wait()
    pltpu.sync_copy(rows_v, o1_hbm.at[pl.ds(base, _BPT)])

    pltpu.sync_copy(id2_hbm.at[pl.ds(base, _BPT)], idx_v)
    pltpu.async_copy(h2_hbm.at[idx_v], rows_v, sem).wait()
    pltpu.sync_copy(rows_v, o2_hbm.at[pl.ds(base, _BPT)])

    pltpu.sync_copy(id3_hbm.at[pl.ds(base, _BPT)], idx_v)
    pltpu.async_copy(ctx_hbm.at[idx_v], ctx_v, sem).wait()
    pltpu.sync_copy(ctx_v, o3_hbm.at[pl.ds(base, _BPT)])


@jax.jit
def kernel(x, edge_index, edge_type, inputs, W_rel1, W_self1, b1,
           W_rel2, W_self2, b2, ctx_emb, Wh1, bh1, Wh2, bh2):
    n, d = x.shape
    src = edge_index[0]
    dst = edge_index[1]

    w_edge = _weights_sc(dst, edge_type)

    w_all1 = jnp.concatenate([W_rel1, W_self1[None]], axis=0)
    hw1 = _rel_matmul(x, w_all1)
    acc1 = _layer_sc(hw1, src, dst, edge_type, w_edge)

    w_all2 = jnp.concatenate([W_rel2, W_self2[None]], axis=0)
    h1, hw2 = _combine_matmul(acc1, hw1, b1, x, w_all2)
    acc2 = _layer_sc(hw2, src, dst, edge_type, w_edge)
    h2 = _combine(acc2, hw2, b2, h1)

    d1, d2, ctx = _head_gather(h2, ctx_emb, inputs[:, 0], inputs[:, 1],
                               inputs[:, 2])
    out = _head_matmul(d1, d2, ctx, Wh1, bh1, Wh2.reshape(1, -1), bh2)
    return out[:, 0]


# speedup vs baseline: 1.0432x; 1.0432x over previous
import functools

import jax
import jax.numpy as jnp
from jax import lax
from jax.experimental import pallas as pl
from jax.experimental.pallas import tpu as pltpu
from jax.experimental.pallas import tpu_sc as plsc

_N = 10000
_E = 320000
_D = 128
_R = 8
_NC = 2
_NS = 16
_K = 80
_G = _K // 16
_NP = 10240
_NPT = _NP // _NS
_CC = _E // (_NS * _K)
_MC = _E // (_NC * _NS * _K)
_ET = _E // (_NC * _NS)

_mesh = plsc.VectorSubcoreMesh(core_axis_name="c", subcore_axis_name="s",
                               num_cores=_NC, num_subcores=_NS)
_sc_params = pltpu.CompilerParams(needs_layout_passes=False,
                                  use_tc_tiling_on_sc=False)


def _rel_matmul(h, w_all):
    n, d = h.shape
    kk = w_all.shape[0]
    blk = 1000
    nb = n // blk

    def body(h_ref, w_ref, o_ref):
        o_ref[...] = jnp.dot(h_ref[...].astype(jnp.bfloat16),
                             w_ref[0].astype(jnp.bfloat16),
                             preferred_element_type=jnp.float32)

    return pl.pallas_call(
        body,
        grid=(kk, nb),
        in_specs=[
            pl.BlockSpec((blk, d), lambda k, b: (b, 0)),
            pl.BlockSpec((1, d, d), lambda k, b: (k, 0, 0)),
        ],
        out_specs=pl.BlockSpec((blk, d), lambda k, b: (k * nb + b, 0)),
        out_shape=jax.ShapeDtypeStruct((kk * n, d), jnp.float32),
    )(h, w_all)


def _combine(accp, hw_full, bias, h_prev):
    n, d = h_prev.shape
    blk = 1000
    nb = n // blk
    self_blk0 = _R * n // blk

    def body(a0, a1, hs, b_ref, hp, o_ref):
        u = jnp.maximum(a0[0] + a1[0] + hs[...] + b_ref[...], 0.0)
        o_ref[...] = u + hp[...]

    row = pl.BlockSpec((blk, d), lambda i: (i, 0))
    return pl.pallas_call(
        body,
        grid=(nb,),
        in_specs=[
            pl.BlockSpec((1, blk, d), lambda i: (0, i, 0)),
            pl.BlockSpec((1, blk, d), lambda i: (1, i, 0)),
            pl.BlockSpec((blk, d), lambda i: (self_blk0 + i, 0)),
            pl.BlockSpec((1, d), lambda i: (0, 0)),
            row,
        ],
        out_specs=row,
        out_shape=jax.ShapeDtypeStruct((n, d), jnp.float32),
    )(accp, accp, hw_full, bias.reshape(1, d), h_prev)


def _combine_matmul(accp, hw1_full, bias, h_prev, w_all2):
    n, d = h_prev.shape
    kk = w_all2.shape[0]
    blk = 1000
    nb = n // blk
    self_blk0 = _R * n // blk

    def body(a0, a1, hs, b_ref, hp, w_ref, h1_ref, o_ref, h1_scr):
        k = pl.program_id(0)
        b = pl.program_id(1)

        @pl.when(k == 0)
        def _():
            u = jnp.maximum(a0[0] + a1[0] + hs[...] + b_ref[...], 0.0)
            h1_scr[pl.ds(b * blk, blk), :] = u + hp[...]

        h1blk = h1_scr[pl.ds(b * blk, blk), :]
        h1_ref[...] = h1blk
        o_ref[...] = jnp.dot(h1blk.astype(jnp.bfloat16),
                             w_ref[0].astype(jnp.bfloat16),
                             preferred_element_type=jnp.float32)

    return pl.pallas_call(
        body,
        grid=(kk, nb),
        in_specs=[
            pl.BlockSpec((1, blk, d), lambda k, b: (0, b, 0)),
            pl.BlockSpec((1, blk, d), lambda k, b: (1, b, 0)),
            pl.BlockSpec((blk, d), lambda k, b: (self_blk0 + b, 0)),
            pl.BlockSpec((1, d), lambda k, b: (0, 0)),
            pl.BlockSpec((blk, d), lambda k, b: (b, 0)),
            pl.BlockSpec((1, d, d), lambda k, b: (k, 0, 0)),
        ],
        out_specs=[
            pl.BlockSpec((blk, d), lambda k, b: (b, 0)),
            pl.BlockSpec((blk, d), lambda k, b: (k * nb + b, 0)),
        ],
        out_shape=[
            jax.ShapeDtypeStruct((n, d), jnp.float32),
            jax.ShapeDtypeStruct((kk * n, d), jnp.float32),
        ],
        scratch_shapes=[pltpu.VMEM((n, d), jnp.float32)],
    )(accp, accp, hw1_full, bias.reshape(1, d), h_prev, w_all2)


def _head_matmul(d1, d2, ctx, wh1, bh1, wh2_row, bh2):
    b, d = d1.shape
    cd = ctx.shape[1]
    h = wh1.shape[1]

    def body(d1r, d2r, cr, w1r, b1r, w2r, b2r, o_ref):
        w1 = w1r[...]
        z = jnp.dot(d1r[...], w1[0:d], preferred_element_type=jnp.float32)
        z = z + jnp.dot(d2r[...], w1[d:2 * d],
                        preferred_element_type=jnp.float32)
        z = z + jnp.dot(cr[...], w1[2 * d:2 * d + cd],
                        preferred_element_type=jnp.float32)
        hdd = jnp.maximum(z + b1r[...], 0.0)
        o_ref[...] = (jnp.sum(hdd * w2r[...], axis=1, keepdims=True)
                      + b2r[...])

    return pl.pallas_call(
        body,
        out_shape=jax.ShapeDtypeStruct((b, 1), jnp.float32),
    )(d1, d2, ctx, wh1, bh1.reshape(1, h), wh2_row, bh2.reshape(1, 1))


def _zero_vmem_rows(ref, rows, width):
    zeros16 = jnp.zeros((16,), jnp.float32)

    @pl.loop(0, rows)
    def _(i):
        for j in range(width // 16):
            ref[i, pl.ds(j * 16, 16)] = zeros16


@functools.partial(
    pl.kernel,
    out_type=jax.ShapeDtypeStruct((_E,), jnp.float32),
    mesh=_mesh,
    compiler_params=_sc_params,
    scratch_types=[
        pltpu.VMEM_SHARED((_NP, 16), jnp.float32),
        [pltpu.VMEM((_K, 16), jnp.float32)] * 2,
        [pltpu.VMEM((_K,), jnp.int32)] * 2,
        [pltpu.VMEM((_K,), jnp.int32)] * 2,
        [pltpu.VMEM((_K,), jnp.int32)] * 2,
        [pltpu.VMEM((_K, 16), jnp.float32)] * 2,
        [pltpu.VMEM((_K,), jnp.float32)] * 2,
        pltpu.VMEM((_NPT, 16), jnp.float32),
        [pltpu.SemaphoreType.DMA] * 2,
        [pltpu.SemaphoreType.DMA] * 2,
        [pltpu.SemaphoreType.DMA] * 2,
    ],
)
def _weights_sc(dst_hbm, typ_hbm, w_hbm,
                cnt_sh, oh_v, dst_v, cdst_v, typ_v, inv_v, w_v, cs_v,
                sem_i, sem_w, sem_o):
    c = lax.axis_index("c")
    s = lax.axis_index("s")
    iota = lax.iota(jnp.int32, 16)
    ones16 = jnp.ones((16,), jnp.float32)
    zeros16 = jnp.zeros((16,), jnp.float32)

    _zero_vmem_rows(cs_v, _NPT, 16)
    pltpu.sync_copy(cs_v, cnt_sh.at[pl.ds(s * _NPT, _NPT)])
    plsc.subcore_barrier()

    def issue_cnt(t, b):
        base = (s * _CC + t) * _K
        pltpu.async_copy(dst_hbm.at[pl.ds(base, _K)], dst_v[b], sem_i[b])
        pltpu.async_copy(typ_hbm.at[pl.ds(base, _K)], typ_v[b], sem_i[b])

    def wait_cnt(t, b):
        base = (s * _CC + t) * _K
        pltpu.make_async_copy(dst_hbm.at[pl.ds(base, _K)], dst_v[b],
                              sem_i[b]).wait()
        pltpu.make_async_copy(typ_hbm.at[pl.ds(base, _K)], typ_v[b],
                              sem_i[b]).wait()

    def do_cnt(t, b):
        @pl.when(t >= 2)
        def _():
            pltpu.make_async_copy(oh_v[b], cnt_sh.at[cdst_v[b]],
                                  sem_o[b]).wait()

        for g in range(_G):
            sl = pl.ds(g * 16, 16)
            cdst_v[b][sl] = dst_v[b][sl]

        @pl.loop(0, _K, unroll=8)
        def _(i):
            tyb = plsc.load_gather(typ_v[b], [jnp.full((16,), i, jnp.int32)])
            oh_v[b][i, pl.ds(0, 16)] = jnp.where(iota == tyb, 1.0, 0.0)

        pltpu.async_copy(oh_v[b], cnt_sh.at[cdst_v[b]], sem_o[b], add=True)

    issue_cnt(0, 0)

    @pl.loop(0, _CC // 2)
    def _(u):
        for binner in range(2):
            ct = 2 * u + binner
            b = binner
            wait_cnt(ct, b)

            @pl.when(ct + 1 < _CC)
            def _():
                issue_cnt(ct + 1, 1 - b)

            do_cnt(ct, b)

    pltpu.make_async_copy(oh_v[0], cnt_sh.at[cdst_v[0]], sem_o[0]).wait()
    pltpu.make_async_copy(oh_v[1], cnt_sh.at[cdst_v[1]], sem_o[1]).wait()
    plsc.subcore_barrier()

    pltpu.sync_copy(cnt_sh.at[pl.ds(s * _NPT, _NPT)], cs_v)

    @pl.loop(0, _NPT)
    def _(i):
        v = cs_v[i, pl.ds(0, 16)]
        cs_v[i, pl.ds(0, 16)] = 1.0 / jnp.maximum(v, 1.0)

    pltpu.sync_copy(cs_v, cnt_sh.at[pl.ds(s * _NPT, _NPT)])
    plsc.subcore_barrier()

    half = c * (_E // _NC) + s * _ET

    def issue_w(t, b):
        base = half + t * _K
        pltpu.async_copy(dst_hbm.at[pl.ds(base, _K)], dst_v[b], sem_i[b])
        pltpu.async_copy(typ_hbm.at[pl.ds(base, _K)], typ_v[b], sem_i[b])

    def wait_w(t, b):
        base = half + t * _K
        pltpu.make_async_copy(dst_hbm.at[pl.ds(base, _K)], dst_v[b],
                              sem_i[b]).wait()
        pltpu.make_async_copy(typ_hbm.at[pl.ds(base, _K)], typ_v[b],
                              sem_i[b]).wait()

    def do_w(t, b):
        pltpu.sync_copy(cnt_sh.at[dst_v[b]], inv_v[b])

        @pl.when(t >= 2)
        def _():
            pltpu.make_async_copy(
                w_v[b], w_hbm.at[pl.ds(half + (t - 2) * _K, _K)],
                sem_w[b]).wait()

        for g in range(_G):
            rows = iota + g * 16
            ty = typ_v[b][pl.ds(g * 16, 16)]
            w_v[b][pl.ds(g * 16, 16)] = plsc.load_gather(inv_v[b], [rows, ty])
        pltpu.async_copy(w_v[b], w_hbm.at[pl.ds(half + t * _K, _K)],
                         sem_w[b])

    issue_w(0, 0)

    @pl.loop(0, _MC // 2)
    def _(u):
        for binner in range(2):
            ct = 2 * u + binner
            b = binner
            wait_w(ct, b)

            @pl.when(ct + 1 < _MC)
            def _():
                issue_w(ct + 1, 1 - b)

            do_w(ct, b)

    wait_w(_MC - 1, 0)
    do_w(_MC - 1, 0)

    pltpu.make_async_copy(w_v[1], w_hbm.at[pl.ds(half + (_MC - 2) * _K, _K)],
                          sem_w[1]).wait()
    pltpu.make_async_copy(w_v[0], w_hbm.at[pl.ds(half + (_MC - 1) * _K, _K)],
                          sem_w[0]).wait()


@functools.partial(
    pl.kernel,
    out_type=jax.ShapeDtypeStruct((_NC, _NP, _D), jnp.float32),
    mesh=_mesh,
    compiler_params=_sc_params,
    scratch_types=[
        pltpu.VMEM_SHARED((_NP, _D), jnp.float32),
        [pltpu.VMEM((_K,), jnp.int32)] * 3,
        [pltpu.VMEM((_K,), jnp.int32)] * 3,
        [pltpu.VMEM((_K,), jnp.int32)] * 3,
        [pltpu.VMEM((_K,), jnp.int32)] * 3,
        [pltpu.VMEM((_K,), jnp.float32)] * 3,
        [pltpu.VMEM((_K, _D), jnp.float32)] * 3,
        [pltpu.SemaphoreType.DMA] * 3,
        [pltpu.SemaphoreType.DMA] * 3,
        [pltpu.SemaphoreType.DMA] * 3,
    ],
)
def _layer_sc(table_hbm, src_hbm, dst_hbm, typ_hbm, w_hbm, acc_hbm,
              acc_sh, dst_v, sdst_v, typ_v, idx_v, w_v, rows_v,
              sem_i, sem_g, sem_s):
    c = lax.axis_index("c")
    s = lax.axis_index("s")

    _zero_vmem_rows(rows_v[0], _K, _D)

    @pl.loop(0, _NPT // _K)
    def _(k):
        pltpu.sync_copy(rows_v[0], acc_sh.at[pl.ds(s * _NPT + k * _K, _K)])

    plsc.subcore_barrier()

    half = c * (_E // _NC) + s * _ET

    def issue_idx(t, b):
        base = half + t * _K
        pltpu.async_copy(src_hbm.at[pl.ds(base, _K)], idx_v[b], sem_i[b])
        pltpu.async_copy(dst_hbm.at[pl.ds(base, _K)], dst_v[b], sem_i[b])
        pltpu.async_copy(typ_hbm.at[pl.ds(base, _K)], typ_v[b], sem_i[b])
        pltpu.async_copy(w_hbm.at[pl.ds(base, _K)], w_v[b], sem_i[b])

    def wait_idx(t, b):
        base = half + t * _K
        pltpu.make_async_copy(src_hbm.at[pl.ds(base, _K)], idx_v[b],
                              sem_i[b]).wait()
        pltpu.make_async_copy(dst_hbm.at[pl.ds(base, _K)], dst_v[b],
                              sem_i[b]).wait()
        pltpu.make_async_copy(typ_hbm.at[pl.ds(base, _K)], typ_v[b],
                              sem_i[b]).wait()
        pltpu.make_async_copy(w_hbm.at[pl.ds(base, _K)], w_v[b],
                              sem_i[b]).wait()

    def start_gather(t, b):
        @pl.when(t >= 3)
        def _():
            pltpu.make_async_copy(rows_v[b], acc_sh.at[sdst_v[b]],
                                  sem_s[b]).wait()

        for g in range(_G):
            sl = pl.ds(g * 16, 16)
            idx_v[b][sl] = typ_v[b][sl] * _N + idx_v[b][sl]
        pltpu.async_copy(table_hbm.at[idx_v[b]], rows_v[b], sem_g[b])

    def finish(b):
        pltpu.make_async_copy(table_hbm.at[idx_v[b]], rows_v[b],
                              sem_g[b]).wait()

        for g in range(_G):
            sl = pl.ds(g * 16, 16)
            sdst_v[b][sl] = dst_v[b][sl]

        @pl.loop(0, _K, unroll=8)
        def _(i):
            wb = plsc.load_gather(w_v[b], [jnp.full((16,), i, jnp.int32)])
            for j in range(_D // 16):
                sl = pl.ds(j * 16, 16)
                rows_v[b][i, sl] = rows_v[b][i, sl] * wb

        pltpu.async_copy(rows_v[b], acc_sh.at[sdst_v[b]], sem_s[b],
                         add=True)

    issue_idx(0, 0)
    wait_idx(0, 0)
    start_gather(0, 0)
    issue_idx(1, 1)
    wait_idx(1, 1)
    start_gather(1, 1)
    issue_idx(2, 2)

    @pl.loop(0, (_MC - 2) // 3)
    def _(u):
        for j in range(3):
            ct = 2 + 3 * u + j
            b = (2 + j) % 3
            wait_idx(ct, b)
            start_gather(ct, b)
            finish(j)

            @pl.when(ct + 1 < _MC)
            def _():
                issue_idx(ct + 1, j)

    finish((_MC - 2) % 3)
    finish((_MC - 1) % 3)

    for b in range(3):
        pltpu.make_async_copy(rows_v[b], acc_sh.at[sdst_v[b]],
                              sem_s[b]).wait()

    plsc.subcore_barrier()

    @pl.loop(0, _NPT // _K)
    def _(k):
        row0 = s * _NPT + k * _K
        pltpu.sync_copy(acc_sh.at[pl.ds(row0, _K)], rows_v[0])
        pltpu.sync_copy(rows_v[0], acc_hbm.at[c, pl.ds(row0, _K)])


_BPT = 1024 // (_NC * _NS)


@functools.partial(
    pl.kernel,
    out_type=[
        jax.ShapeDtypeStruct((1024, _D), jnp.float32),
        jax.ShapeDtypeStruct((1024, _D), jnp.float32),
        jax.ShapeDtypeStruct((1024, 64), jnp.float32),
    ],
    mesh=_mesh,
    compiler_params=_sc_params,
    scratch_types=[
        pltpu.VMEM((_BPT,), jnp.int32),
        pltpu.VMEM((_BPT, _D), jnp.float32),
        pltpu.VMEM((_BPT, 64), jnp.float32),
        pltpu.SemaphoreType.DMA,
    ],
)
def _head_gather(h2_hbm, ctx_hbm, id1_hbm, id2_hbm, id3_hbm,
                 o1_hbm, o2_hbm, o3_hbm, idx_v, rows_v, ctx_v, sem):
    c = lax.axis_index("c")
    s = lax.axis_index("s")
    base = (s * _NC + c) * _BPT

    pltpu.sync_copy(id1_hbm.at[pl.ds(base, _BPT)], idx_v)
    pltpu.async_copy(h2_hbm.at[idx_v], rows_v, sem).wait()
    pltpu.sync_copy(rows_v, o1_hbm.at[pl.ds(base, _BPT)])

    pltpu.sync_copy(id2_hbm.at[pl.ds(base, _BPT)], idx_v)
    pltpu.async_copy(h2_hbm.at[idx_v], rows_v, sem).wait()
    pltpu.sync_copy(rows_v, o2_hbm.at[pl.ds(base, _BPT)])

    pltpu.sync_copy(id3_hbm.at[pl.ds(base, _BPT)], idx_v)
    pltpu.async_copy(ctx_hbm.at[idx_v], ctx_v, sem).wait()
    pltpu.sync_copy(ctx_v, o3_hbm.at[pl.ds(base, _BPT)])


@jax.jit
def kernel(x, edge_index, edge_type, inputs, W_rel1, W_self1, b1,
           W_rel2, W_self2, b2, ctx_emb, Wh1, bh1, Wh2, bh2):
    n, d = x.shape
    src = edge_index[0]
    dst = edge_index[1]

    w_edge = _weights_sc(dst, edge_type)

    w_all1 = jnp.concatenate([W_rel1, W_self1[None]], axis=0)
    hw1 = _rel_matmul(x, w_all1)
    acc1 = _layer_sc(hw1, src, dst, edge_type, w_edge)

    h1 = _combine(acc1, hw1, b1, x)

    w_all2 = jnp.concatenate([W_rel2, W_self2[None]], axis=0)
    hw2 = _rel_matmul(h1, w_all2)
    acc2 = _layer_sc(hw2, src, dst, edge_type, w_edge)
    h2 = _combine(acc2, hw2, b2, h1)

    d1, d2, ctx = _head_gather(h2, ctx_emb, inputs[:, 0], inputs[:, 1],
                               inputs[:, 2])
    out = _head_matmul(d1, d2, ctx, Wh1, bh1, Wh2.reshape(1, -1), bh2)
    return out[:, 0]


# speedup vs baseline: 1.0472x; 1.0038x over previous
import functools

import jax
import jax.numpy as jnp
from jax import lax
from jax.experimental import pallas as pl
from jax.experimental.pallas import tpu as pltpu
from jax.experimental.pallas import tpu_sc as plsc

_N = 10000
_E = 320000
_D = 128
_R = 8
_NC = 2
_NS = 16
_K = 80
_G = _K // 16
_NP = 10240
_NPT = _NP // _NS
_CC = _E // (_NS * _K)
_MC = _E // (_NC * _NS * _K)
_ET = _E // (_NC * _NS)

_mesh = plsc.VectorSubcoreMesh(core_axis_name="c", subcore_axis_name="s",
                               num_cores=_NC, num_subcores=_NS)
_sc_params = pltpu.CompilerParams(needs_layout_passes=False,
                                  use_tc_tiling_on_sc=False)


def _rel_matmul(h, w_all):
    n, d = h.shape
    kk = w_all.shape[0]
    blk = 1000
    nb = n // blk

    def body(h_ref, w_ref, o_ref):
        o_ref[...] = jnp.dot(h_ref[...].astype(jnp.bfloat16),
                             w_ref[0].astype(jnp.bfloat16),
                             preferred_element_type=jnp.float32)

    return pl.pallas_call(
        body,
        grid=(kk, nb),
        in_specs=[
            pl.BlockSpec((blk, d), lambda k, b: (b, 0)),
            pl.BlockSpec((1, d, d), lambda k, b: (k, 0, 0)),
        ],
        out_specs=pl.BlockSpec((blk, d), lambda k, b: (k * nb + b, 0)),
        out_shape=jax.ShapeDtypeStruct((kk * n, d), jnp.float32),
    )(h, w_all)


def _combine(accp, hw_full, bias, h_prev):
    n, d = h_prev.shape
    blk = 1000
    nb = n // blk
    self_blk0 = _R * n // blk

    def body(a0, a1, hs, b_ref, hp, o_ref):
        u = jnp.maximum(a0[0] + a1[0] + hs[...] + b_ref[...], 0.0)
        o_ref[...] = u + hp[...]

    row = pl.BlockSpec((blk, d), lambda i: (i, 0))
    return pl.pallas_call(
        body,
        grid=(nb,),
        in_specs=[
            pl.BlockSpec((1, blk, d), lambda i: (0, i, 0)),
            pl.BlockSpec((1, blk, d), lambda i: (1, i, 0)),
            pl.BlockSpec((blk, d), lambda i: (self_blk0 + i, 0)),
            pl.BlockSpec((1, d), lambda i: (0, 0)),
            row,
        ],
        out_specs=row,
        out_shape=jax.ShapeDtypeStruct((n, d), jnp.float32),
    )(accp, accp, hw_full, bias.reshape(1, d), h_prev)


def _combine_matmul(accp, hw1_full, bias, h_prev, w_all2):
    n, d = h_prev.shape
    kk = w_all2.shape[0]
    blk = 1000
    nb = n // blk
    self_blk0 = _R * n // blk

    def body(a0, a1, hs, b_ref, hp, w_ref, h1_ref, o_ref, h1_scr):
        k = pl.program_id(0)
        b = pl.program_id(1)

        @pl.when(k == 0)
        def _():
            u = jnp.maximum(a0[0] + a1[0] + hs[...] + b_ref[...], 0.0)
            h1_scr[pl.ds(b * blk, blk), :] = u + hp[...]

        h1blk = h1_scr[pl.ds(b * blk, blk), :]
        h1_ref[...] = h1blk
        o_ref[...] = jnp.dot(h1blk.astype(jnp.bfloat16),
                             w_ref[0].astype(jnp.bfloat16),
                             preferred_element_type=jnp.float32)

    return pl.pallas_call(
        body,
        grid=(kk, nb),
        in_specs=[
            pl.BlockSpec((1, blk, d), lambda k, b: (0, b, 0)),
            pl.BlockSpec((1, blk, d), lambda k, b: (1, b, 0)),
            pl.BlockSpec((blk, d), lambda k, b: (self_blk0 + b, 0)),
            pl.BlockSpec((1, d), lambda k, b: (0, 0)),
            pl.BlockSpec((blk, d), lambda k, b: (b, 0)),
            pl.BlockSpec((1, d, d), lambda k, b: (k, 0, 0)),
        ],
        out_specs=[
            pl.BlockSpec((blk, d), lambda k, b: (b, 0)),
            pl.BlockSpec((blk, d), lambda k, b: (k * nb + b, 0)),
        ],
        out_shape=[
            jax.ShapeDtypeStruct((n, d), jnp.float32),
            jax.ShapeDtypeStruct((kk * n, d), jnp.float32),
        ],
        scratch_shapes=[pltpu.VMEM((n, d), jnp.float32)],
    )(accp, accp, hw1_full, bias.reshape(1, d), h_prev, w_all2)


def _head_matmul(z1, r1, z2, r2, ctx, b2v, wh1, bh1, wh2_row, bh2):
    b, d = z1.shape
    cd = ctx.shape[1]
    h = wh1.shape[1]

    def body(z1r, r1r, z2r, r2r, cr, b2r, w1r, b1r, w2r, b2hr, o_ref):
        d1 = jnp.maximum(z1r[...] + b2r[...], 0.0) + r1r[...]
        d2 = jnp.maximum(z2r[...] + b2r[...], 0.0) + r2r[...]
        w1 = w1r[...]
        z = jnp.dot(d1, w1[0:d], preferred_element_type=jnp.float32)
        z = z + jnp.dot(d2, w1[d:2 * d],
                        preferred_element_type=jnp.float32)
        z = z + jnp.dot(cr[...], w1[2 * d:2 * d + cd],
                        preferred_element_type=jnp.float32)
        hdd = jnp.maximum(z + b1r[...], 0.0)
        o_ref[...] = (jnp.sum(hdd * w2r[...], axis=1, keepdims=True)
                      + b2hr[...])

    return pl.pallas_call(
        body,
        out_shape=jax.ShapeDtypeStruct((b, 1), jnp.float32),
    )(z1, r1, z2, r2, ctx, b2v.reshape(1, d), wh1, bh1.reshape(1, h),
      wh2_row, bh2.reshape(1, 1))


def _zero_vmem_rows(ref, rows, width):
    zeros16 = jnp.zeros((16,), jnp.float32)

    @pl.loop(0, rows)
    def _(i):
        for j in range(width // 16):
            ref[i, pl.ds(j * 16, 16)] = zeros16


@functools.partial(
    pl.kernel,
    out_type=jax.ShapeDtypeStruct((_E,), jnp.float32),
    mesh=_mesh,
    compiler_params=_sc_params,
    scratch_types=[
        pltpu.VMEM_SHARED((_NP, 16), jnp.float32),
        [pltpu.VMEM((_K, 16), jnp.float32)] * 2,
        [pltpu.VMEM((_K,), jnp.int32)] * 2,
        [pltpu.VMEM((_K,), jnp.int32)] * 2,
        [pltpu.VMEM((_K,), jnp.int32)] * 2,
        [pltpu.VMEM((_K, 16), jnp.float32)] * 2,
        [pltpu.VMEM((_K,), jnp.float32)] * 2,
        pltpu.VMEM((_NPT, 16), jnp.float32),
        [pltpu.SemaphoreType.DMA] * 2,
        [pltpu.SemaphoreType.DMA] * 2,
        [pltpu.SemaphoreType.DMA] * 2,
    ],
)
def _weights_sc(dst_hbm, typ_hbm, w_hbm,
                cnt_sh, oh_v, dst_v, cdst_v, typ_v, inv_v, w_v, cs_v,
                sem_i, sem_w, sem_o):
    c = lax.axis_index("c")
    s = lax.axis_index("s")
    iota = lax.iota(jnp.int32, 16)
    ones16 = jnp.ones((16,), jnp.float32)
    zeros16 = jnp.zeros((16,), jnp.float32)

    _zero_vmem_rows(cs_v, _NPT, 16)
    pltpu.sync_copy(cs_v, cnt_sh.at[pl.ds(s * _NPT, _NPT)])
    plsc.subcore_barrier()

    def issue_cnt(t, b):
        base = (s * _CC + t) * _K
        pltpu.async_copy(dst_hbm.at[pl.ds(base, _K)], dst_v[b], sem_i[b])
        pltpu.async_copy(typ_hbm.at[pl.ds(base, _K)], typ_v[b], sem_i[b])

    def wait_cnt(t, b):
        base = (s * _CC + t) * _K
        pltpu.make_async_copy(dst_hbm.at[pl.ds(base, _K)], dst_v[b],
                              sem_i[b]).wait()
        pltpu.make_async_copy(typ_hbm.at[pl.ds(base, _K)], typ_v[b],
                              sem_i[b]).wait()

    def do_cnt(t, b):
        @pl.when(t >= 2)
        def _():
            pltpu.make_async_copy(oh_v[b], cnt_sh.at[cdst_v[b]],
                                  sem_o[b]).wait()

        for g in range(_G):
            sl = pl.ds(g * 16, 16)
            cdst_v[b][sl] = dst_v[b][sl]

        @pl.loop(0, _K, unroll=8)
        def _(i):
            tyb = plsc.load_gather(typ_v[b], [jnp.full((16,), i, jnp.int32)])
            oh_v[b][i, pl.ds(0, 16)] = jnp.where(iota == tyb, 1.0, 0.0)

        pltpu.async_copy(oh_v[b], cnt_sh.at[cdst_v[b]], sem_o[b], add=True)

    issue_cnt(0, 0)

    @pl.loop(0, _CC // 2)
    def _(u):
        for binner in range(2):
            ct = 2 * u + binner
            b = binner
            wait_cnt(ct, b)

            @pl.when(ct + 1 < _CC)
            def _():
                issue_cnt(ct + 1, 1 - b)

            do_cnt(ct, b)

    pltpu.make_async_copy(oh_v[0], cnt_sh.at[cdst_v[0]], sem_o[0]).wait()
    pltpu.make_async_copy(oh_v[1], cnt_sh.at[cdst_v[1]], sem_o[1]).wait()
    plsc.subcore_barrier()

    pltpu.sync_copy(cnt_sh.at[pl.ds(s * _NPT, _NPT)], cs_v)

    @pl.loop(0, _NPT)
    def _(i):
        v = cs_v[i, pl.ds(0, 16)]
        cs_v[i, pl.ds(0, 16)] = 1.0 / jnp.maximum(v, 1.0)

    pltpu.sync_copy(cs_v, cnt_sh.at[pl.ds(s * _NPT, _NPT)])
    plsc.subcore_barrier()

    half = c * (_E // _NC) + s * _ET

    def issue_w(t, b):
        base = half + t * _K
        pltpu.async_copy(dst_hbm.at[pl.ds(base, _K)], dst_v[b], sem_i[b])
        pltpu.async_copy(typ_hbm.at[pl.ds(base, _K)], typ_v[b], sem_i[b])

    def wait_w(t, b):
        base = half + t * _K
        pltpu.make_async_copy(dst_hbm.at[pl.ds(base, _K)], dst_v[b],
                              sem_i[b]).wait()
        pltpu.make_async_copy(typ_hbm.at[pl.ds(base, _K)], typ_v[b],
                              sem_i[b]).wait()

    def do_w(t, b):
        pltpu.sync_copy(cnt_sh.at[dst_v[b]], inv_v[b])

        @pl.when(t >= 2)
        def _():
            pltpu.make_async_copy(
                w_v[b], w_hbm.at[pl.ds(half + (t - 2) * _K, _K)],
                sem_w[b]).wait()

        for g in range(_G):
            rows = iota + g * 16
            ty = typ_v[b][pl.ds(g * 16, 16)]
            w_v[b][pl.ds(g * 16, 16)] = plsc.load_gather(inv_v[b], [rows, ty])
        pltpu.async_copy(w_v[b], w_hbm.at[pl.ds(half + t * _K, _K)],
                         sem_w[b])

    issue_w(0, 0)

    @pl.loop(0, _MC // 2)
    def _(u):
        for binner in range(2):
            ct = 2 * u + binner
            b = binner
            wait_w(ct, b)

            @pl.when(ct + 1 < _MC)
            def _():
                issue_w(ct + 1, 1 - b)

            do_w(ct, b)

    wait_w(_MC - 1, 0)
    do_w(_MC - 1, 0)

    pltpu.make_async_copy(w_v[1], w_hbm.at[pl.ds(half + (_MC - 2) * _K, _K)],
                          sem_w[1]).wait()
    pltpu.make_async_copy(w_v[0], w_hbm.at[pl.ds(half + (_MC - 1) * _K, _K)],
                          sem_w[0]).wait()


@functools.partial(
    pl.kernel,
    out_type=jax.ShapeDtypeStruct((_NC, _NP, _D), jnp.float32),
    mesh=_mesh,
    compiler_params=_sc_params,
    scratch_types=[
        pltpu.VMEM_SHARED((_NP, _D), jnp.float32),
        [pltpu.VMEM((_K,), jnp.int32)] * 3,
        [pltpu.VMEM((_K,), jnp.int32)] * 3,
        [pltpu.VMEM((_K,), jnp.int32)] * 3,
        [pltpu.VMEM((_K,), jnp.int32)] * 3,
        [pltpu.VMEM((_K,), jnp.float32)] * 3,
        [pltpu.VMEM((_K, _D), jnp.float32)] * 3,
        [pltpu.SemaphoreType.DMA] * 3,
        [pltpu.SemaphoreType.DMA] * 3,
        [pltpu.SemaphoreType.DMA] * 3,
    ],
)
def _layer_sc(table_hbm, src_hbm, dst_hbm, typ_hbm, w_hbm, acc_hbm,
              acc_sh, dst_v, sdst_v, typ_v, idx_v, w_v, rows_v,
              sem_i, sem_g, sem_s):
    c = lax.axis_index("c")
    s = lax.axis_index("s")

    _zero_vmem_rows(rows_v[0], _K, _D)

    @pl.loop(0, _NPT // _K)
    def _(k):
        pltpu.sync_copy(rows_v[0], acc_sh.at[pl.ds(s * _NPT + k * _K, _K)])

    plsc.subcore_barrier()

    half = c * (_E // _NC) + s * _ET

    def issue_idx(t, b):
        base = half + t * _K
        pltpu.async_copy(src_hbm.at[pl.ds(base, _K)], idx_v[b], sem_i[b])
        pltpu.async_copy(dst_hbm.at[pl.ds(base, _K)], dst_v[b], sem_i[b])
        pltpu.async_copy(typ_hbm.at[pl.ds(base, _K)], typ_v[b], sem_i[b])
        pltpu.async_copy(w_hbm.at[pl.ds(base, _K)], w_v[b], sem_i[b])

    def wait_idx(t, b):
        base = half + t * _K
        pltpu.make_async_copy(src_hbm.at[pl.ds(base, _K)], idx_v[b],
                              sem_i[b]).wait()
        pltpu.make_async_copy(dst_hbm.at[pl.ds(base, _K)], dst_v[b],
                              sem_i[b]).wait()
        pltpu.make_async_copy(typ_hbm.at[pl.ds(base, _K)], typ_v[b],
                              sem_i[b]).wait()
        pltpu.make_async_copy(w_hbm.at[pl.ds(base, _K)], w_v[b],
                              sem_i[b]).wait()

    def start_gather(t, b):
        @pl.when(t >= 3)
        def _():
            pltpu.make_async_copy(rows_v[b], acc_sh.at[sdst_v[b]],
                                  sem_s[b]).wait()

        for g in range(_G):
            sl = pl.ds(g * 16, 16)
            idx_v[b][sl] = typ_v[b][sl] * _N + idx_v[b][sl]
        pltpu.async_copy(table_hbm.at[idx_v[b]], rows_v[b], sem_g[b])

    def finish(b):
        pltpu.make_async_copy(table_hbm.at[idx_v[b]], rows_v[b],
                              sem_g[b]).wait()

        for g in range(_G):
            sl = pl.ds(g * 16, 16)
            sdst_v[b][sl] = dst_v[b][sl]

        @pl.loop(0, _K, unroll=8)
        def _(i):
            wb = plsc.load_gather(w_v[b], [jnp.full((16,), i, jnp.int32)])
            for j in range(_D // 16):
                sl = pl.ds(j * 16, 16)
                rows_v[b][i, sl] = rows_v[b][i, sl] * wb

        pltpu.async_copy(rows_v[b], acc_sh.at[sdst_v[b]], sem_s[b],
                         add=True)

    issue_idx(0, 0)
    wait_idx(0, 0)
    start_gather(0, 0)
    issue_idx(1, 1)
    wait_idx(1, 1)
    start_gather(1, 1)
    issue_idx(2, 2)

    @pl.loop(0, (_MC - 2) // 3)
    def _(u):
        for j in range(3):
            ct = 2 + 3 * u + j
            b = (2 + j) % 3
            wait_idx(ct, b)
            start_gather(ct, b)
            finish(j)

            @pl.when(ct + 1 < _MC)
            def _():
                issue_idx(ct + 1, j)

    finish((_MC - 2) % 3)
    finish((_MC - 1) % 3)

    for b in range(3):
        pltpu.make_async_copy(rows_v[b], acc_sh.at[sdst_v[b]],
                              sem_s[b]).wait()

    plsc.subcore_barrier()

    @pl.loop(0, _NPT // _K)
    def _(k):
        row0 = s * _NPT + k * _K
        pltpu.sync_copy(acc_sh.at[pl.ds(row0, _K)], rows_v[0])
        pltpu.sync_copy(rows_v[0], acc_hbm.at[c, pl.ds(row0, _K)])


_BPT = 1024 // (_NC * _NS)


@functools.partial(
    pl.kernel,
    out_type=[
        jax.ShapeDtypeStruct((1024, _D), jnp.float32),
        jax.ShapeDtypeStruct((1024, _D), jnp.float32),
        jax.ShapeDtypeStruct((1024, _D), jnp.float32),
        jax.ShapeDtypeStruct((1024, _D), jnp.float32),
        jax.ShapeDtypeStruct((1024, 64), jnp.float32),
    ],
    mesh=_mesh,
    compiler_params=_sc_params,
    scratch_types=[
        pltpu.VMEM((_BPT,), jnp.int32),
        pltpu.VMEM((_BPT,), jnp.int32),
        pltpu.VMEM((_BPT, _D), jnp.float32),
        pltpu.VMEM((_BPT, _D), jnp.float32),
        pltpu.VMEM((_BPT, 64), jnp.float32),
        pltpu.SemaphoreType.DMA,
    ],
)
def _head_gather(accf_hbm, hw2_hbm, h1_hbm, ctx_hbm, id1_hbm, id2_hbm,
                 id3_hbm, z1_hbm, r1_hbm, z2_hbm, r2_hbm, o3_hbm,
                 idx_v, idx2_v, rows_v, sum_v, ctx_v, sem):
    c = lax.axis_index("c")
    s = lax.axis_index("s")
    base = (s * _NC + c) * _BPT

    def addin():
        @pl.loop(0, _BPT)
        def _(i):
            for j in range(_D // 16):
                sl = pl.ds(j * 16, 16)
                sum_v[i, sl] = sum_v[i, sl] + rows_v[i, sl]

    def one_drug(id_hbm, z_hbm, r_hbm):
        pltpu.sync_copy(id_hbm.at[pl.ds(base, _BPT)], idx_v)
        pltpu.async_copy(accf_hbm.at[idx_v], sum_v, sem).wait()
        for g in range(_BPT // 16):
            sl = pl.ds(g * 16, 16)
            idx2_v[sl] = idx_v[sl] + _NP
        pltpu.async_copy(accf_hbm.at[idx2_v], rows_v, sem).wait()
        addin()
        for g in range(_BPT // 16):
            sl = pl.ds(g * 16, 16)
            idx2_v[sl] = idx_v[sl] + _R * _N
        pltpu.async_copy(hw2_hbm.at[idx2_v], rows_v, sem).wait()
        addin()
        pltpu.sync_copy(sum_v, z_hbm.at[pl.ds(base, _BPT)])
        pltpu.async_copy(h1_hbm.at[idx_v], rows_v, sem).wait()
        pltpu.sync_copy(rows_v, r_hbm.at[pl.ds(base, _BPT)])

    one_drug(id1_hbm, z1_hbm, r1_hbm)
    one_drug(id2_hbm, z2_hbm, r2_hbm)

    pltpu.sync_copy(id3_hbm.at[pl.ds(base, _BPT)], idx_v)
    pltpu.async_copy(ctx_hbm.at[idx_v], ctx_v, sem).wait()
    pltpu.sync_copy(ctx_v, o3_hbm.at[pl.ds(base, _BPT)])


@jax.jit
def kernel(x, edge_index, edge_type, inputs, W_rel1, W_self1, b1,
           W_rel2, W_self2, b2, ctx_emb, Wh1, bh1, Wh2, bh2):
    n, d = x.shape
    src = edge_index[0]
    dst = edge_index[1]

    w_edge = _weights_sc(dst, edge_type)

    w_all1 = jnp.concatenate([W_rel1, W_self1[None]], axis=0)
    hw1 = _rel_matmul(x, w_all1)
    acc1 = _layer_sc(hw1, src, dst, edge_type, w_edge)

    h1 = _combine(acc1, hw1, b1, x)

    w_all2 = jnp.concatenate([W_rel2, W_self2[None]], axis=0)
    hw2 = _rel_matmul(h1, w_all2)
    acc2 = _layer_sc(hw2, src, dst, edge_type, w_edge)

    z1, r1, z2, r2, ctx = _head_gather(acc2.reshape(2 * _NP, d), hw2, h1,
                                       ctx_emb, inputs[:, 0], inputs[:, 1],
                                       inputs[:, 2])
    out = _head_matmul(z1, r1, z2, r2, ctx, b2, Wh1, bh1,
                       Wh2.reshape(1, -1), bh2)
    return out[:, 0]
